# Initial kernel scaffold; baseline (speedup 1.0000x reference)
import jax
import jax.numpy as jnp
from jax.experimental import pallas as pl


def kernel(x, edge_index, batch, W_pre, b_pre, W_emb, b_emb, W_asg, b_asg, W1, b1, W2, b2):
    raise NotImplementedError("write your pallas kernel here")


# speedup vs baseline: 20.0316x; 20.0316x over previous
import functools

import jax
import jax.numpy as jnp
from jax import lax
from jax.experimental import pallas as pl
from jax.experimental.pallas import tpu as pltpu
from jax.experimental.pallas import tpu_sc as plsc

N = 10000
E = 320000
D = 128
K = 64
B = 50
C = 10

NC = 2
NS = 16
CH = 128
NCHUNK = E // CH
ITERS = -(-NCHUNK // (NC * NS))
NP = 10240
RPT = NP // NS

_mesh = plsc.VectorSubcoreMesh(core_axis_name="c", subcore_axis_name="s")


@functools.partial(
    pl.kernel,
    out_type=jax.ShapeDtypeStruct((NC, NP, 16), jnp.float32),
    mesh=_mesh,
    scratch_types=[
        pltpu.VMEM((CH, 16), jnp.float32),
        pltpu.VMEM((CH,), jnp.int32),
        pltpu.VMEM_SHARED((NP, 16), jnp.float32),
    ],
)
def _deg_kernel(col_hbm, ones_hbm, zeros_hbm, out_hbm, ones_v, cidx_v, acc):
    c = lax.axis_index("c")
    s = lax.axis_index("s")
    w = c * NS + s
    pltpu.sync_copy(ones_hbm, ones_v)
    pltpu.sync_copy(zeros_hbm.at[pl.ds(s * RPT, RPT)], acc.at[pl.ds(s * RPT, RPT)])
    plsc.subcore_barrier()

    def body(j, carry):
        cid = w + j * (NC * NS)

        @pl.when(cid < NCHUNK)
        def _():
            pltpu.sync_copy(col_hbm.at[pl.ds(cid * CH, CH)], cidx_v)
            pltpu.sync_copy(ones_v, acc.at[cidx_v], add=True)
        return carry

    lax.fori_loop(0, ITERS, body, 0)
    plsc.subcore_barrier()
    pltpu.sync_copy(acc.at[pl.ds(s * RPT, RPT)],
                    out_hbm.at[c, pl.ds(s * RPT, RPT)])


@functools.partial(
    pl.kernel,
    out_type=jax.ShapeDtypeStruct((NC, NP, D), jnp.float32),
    mesh=_mesh,
    scratch_types=[
        pltpu.VMEM((CH,), jnp.int32),
        pltpu.VMEM((CH,), jnp.int32),
        pltpu.VMEM((CH, D), jnp.float32),
        pltpu.VMEM_SHARED((NP, D), jnp.float32),
        pltpu.SemaphoreType.DMA,
    ],
)
def _agg_kernel(y_hbm, row_hbm, col_hbm, zeros_hbm, out_hbm,
                ridx_v, cidx_v, rows_v, acc, sem):
    c = lax.axis_index("c")
    s = lax.axis_index("s")
    w = c * NS + s
    pltpu.sync_copy(zeros_hbm.at[pl.ds(s * RPT, RPT)], acc.at[pl.ds(s * RPT, RPT)])
    plsc.subcore_barrier()

    def body(j, carry):
        cid = w + j * (NC * NS)

        @pl.when(cid < NCHUNK)
        def _():
            pltpu.sync_copy(row_hbm.at[pl.ds(cid * CH, CH)], ridx_v)
            pltpu.sync_copy(col_hbm.at[pl.ds(cid * CH, CH)], cidx_v)
            pltpu.async_copy(y_hbm.at[ridx_v], rows_v, sem).wait()
            pltpu.sync_copy(rows_v, acc.at[cidx_v], add=True)
        return carry

    lax.fori_loop(0, ITERS, body, 0)
    plsc.subcore_barrier()
    pltpu.sync_copy(acc.at[pl.ds(s * RPT, RPT)],
                    out_hbm.at[c, pl.ds(s * RPT, RPT)])


def _t0_body(x_ref, w_ref, degp_ref, y1_ref, dinv_ref):
    deg = degp_ref[0, :N, 0:1] + degp_ref[1, :N, 0:1] + 1.0
    dinv = lax.rsqrt(deg)
    xw = jnp.dot(x_ref[...], w_ref[...], preferred_element_type=jnp.float32)
    dinv_ref[...] = dinv
    y1_ref[...] = dinv * xw


def _t1_body(zp_ref, y1_ref, dinv_ref, b_ref, w_ref, y2_ref):
    dinv = dinv_ref[...]
    h = jnp.maximum(dinv * (zp_ref[0, :N] + zp_ref[1, :N] + y1_ref[...])
                    + b_ref[...], 0.0)
    y2_ref[...] = dinv * jnp.dot(h, w_ref[...],
                                 preferred_element_type=jnp.float32)


def _t2_body(zp_ref, y2_ref, dinv_ref, b_ref, batch_ref, w1_ref, b1_ref,
             w2_ref, b2_ref, out_ref):
    dinv = dinv_ref[...]
    ne = jnp.maximum(dinv * (zp_ref[0, :N] + zp_ref[1, :N] + y2_ref[...])
                     + b_ref[...], 0.0)
    labels = lax.broadcasted_iota(jnp.int32, (N, K), 1)
    mask = jnp.where(batch_ref[...] == labels, jnp.float32(1.0 / K),
                     jnp.float32(0.0))
    pooled = lax.dot_general(mask, ne, (((0,), (0,)), ((), ())),
                             preferred_element_type=jnp.float32)
    t = jnp.maximum(jnp.dot(pooled, w1_ref[...],
                            preferred_element_type=jnp.float32) + b1_ref[...],
                    0.0)
    out_ref[...] = jnp.dot(t, w2_ref[...],
                           preferred_element_type=jnp.float32) + b2_ref[...]


_t0 = pl.pallas_call(
    _t0_body,
    out_shape=(jax.ShapeDtypeStruct((N, D), jnp.float32),
               jax.ShapeDtypeStruct((N, 1), jnp.float32)),
)
_t1 = pl.pallas_call(
    _t1_body,
    out_shape=jax.ShapeDtypeStruct((N, D), jnp.float32),
)
_t2 = pl.pallas_call(
    _t2_body,
    out_shape=jax.ShapeDtypeStruct((K, C), jnp.float32),
)


def kernel(x, edge_index, batch, W_pre, b_pre, W_emb, b_emb, W_asg, b_asg,
           W1, b1, W2, b2):
    row = edge_index[0]
    col = edge_index[1]
    zeros_d = jnp.zeros((NP, D), jnp.float32)
    zeros16 = jnp.zeros((NP, 16), jnp.float32)
    ones16 = jnp.ones((CH, 16), jnp.float32)

    degp = _deg_kernel(col, ones16, zeros16)
    y1, dinv = _t0(x, W_pre, degp)
    z1 = _agg_kernel(y1, row, col, zeros_d)
    y2 = _t1(z1, y1, dinv, b_pre.reshape(1, D), W_emb)
    z2 = _agg_kernel(y2, row, col, zeros_d)
    out = _t2(z2, y2, dinv, b_emb.reshape(1, D), batch.reshape(N, 1),
              W1, b1.reshape(1, D), W2, b2.reshape(1, C))
    return out[:B]
